# baseline (speedup 1.0000x reference)
import functools

import jax
import jax.numpy as jnp
from jax import lax
from jax.experimental import pallas as pl
from jax.experimental.pallas import tpu as pltpu
from jax.experimental.pallas import tpu_sc as plsc

_K = 8192
_D = 32
_IN = 256
_CC = 0.25
_EPS = 1e-5
_TB = 1024
_KC = 8
_TB2 = 512
_NW = 32
_T = 4096
_BPW = _T // _NW


def _dist_body(x_ref, emb_ref, w1_ref, b1_ref, bn1w_ref, bn1b_ref,
               idx_ref, bsq_ref):
    i = pl.program_id(0)

    @pl.when(i == 0)
    def _precompute():
        e = emb_ref[...]
        bsq_ref[...] = jnp.sum(e * e, axis=1)[None, :]

    xb = x_ref[...]
    sq = jnp.sqrt(jnp.float32(1.0 + _EPS))
    flat = (xb / sq) * bn1w_ref[...] + bn1b_ref[...]
    h = lax.dot_general(flat, w1_ref[...], (((1,), (1,)), ((), ())),
                        preferred_element_type=jnp.float32) + b1_ref[...]
    a = jnp.sum(h * h, axis=1, keepdims=True)
    kc = _K // _KC
    minv = None
    idx = None
    for j in range(_KC):
        cj = lax.dot_general(h, emb_ref[pl.ds(j * kc, kc), :],
                             (((1,), (1,)), ((), ())),
                             preferred_element_type=jnp.float32)
        dj = (a + bsq_ref[:, pl.ds(j * kc, kc)]) - 2.0 * cj
        mj = jnp.min(dj, axis=1, keepdims=True)
        io = lax.broadcasted_iota(jnp.int32, (_TB, kc), 1) + j * kc
        ij = jnp.min(jnp.where(dj == mj, io, _K), axis=1)
        if j == 0:
            minv, idx = mj, ij
        else:
            take = (mj < minv)[:, 0]
            idx = jnp.where(take, ij, idx)
            minv = jnp.minimum(minv, mj)
    idx_ref[...] = idx.reshape(1, 1, _TB)


_sc_mesh = plsc.VectorSubcoreMesh(core_axis_name="c", subcore_axis_name="s")


@functools.partial(
    pl.kernel,
    out_type=[
        jax.ShapeDtypeStruct((_T, 128), jnp.float32),
        jax.ShapeDtypeStruct((2, _K), jnp.float32),
    ],
    mesh=_sc_mesh,
    scratch_types=[
        pltpu.VMEM((_BPW,), jnp.int32),
        pltpu.VMEM((_BPW, 128), jnp.float32),
        pltpu.VMEM((_BPW,), jnp.float32),
        pltpu.VMEM((_K,), jnp.float32),
        pltpu.VMEM_SHARED((_K,), jnp.float32),
        pltpu.SemaphoreType.DMA,
    ],
)
def _sc_gather_hist(idx_hbm, emb_hbm, q_hbm, hist_hbm,
                    idx_v, rows_v, ones_v, zbuf_v, shared, sem):
    s = lax.axis_index("s")
    c = lax.axis_index("c")
    wid = s * 2 + c
    base = wid * _BPW
    pltpu.sync_copy(idx_hbm.at[pl.ds(base, _BPW)], idx_v)
    gather = pltpu.async_copy(emb_hbm.at[idx_v], rows_v, sem)

    zeros16 = jnp.zeros((16,), jnp.float32)
    ones16 = jnp.ones((16,), jnp.float32)

    @pl.when(s == 0)
    def _zero_shared():
        def _zero(j, carry):
            zbuf_v[pl.ds(j * 16, 16)] = zeros16
            return carry

        lax.fori_loop(0, _K // 16, _zero, 0)
        pltpu.sync_copy(zbuf_v, shared)

    for j in range(_BPW // 16):
        ones_v[pl.ds(j * 16, 16)] = ones16
    plsc.subcore_barrier()
    pltpu.sync_copy(ones_v, shared.at[idx_v], add=True)
    plsc.subcore_barrier()

    @pl.when(s == 0)
    def _dump_hist():
        pltpu.sync_copy(shared, hist_hbm.at[c])

    gather.wait()
    pltpu.sync_copy(rows_v, q_hbm.at[pl.ds(base, _BPW)])


def _out_body(x_ref, q_ref, w2_ref, b2_ref, bn2w_ref, bn2b_ref, hist_ref,
              qst_ref, counts_ref, losssum_ref):
    i = pl.program_id(0)
    xb = x_ref[...]
    sq = jnp.sqrt(jnp.float32(1.0 + _EPS))
    qbn = (q_ref[:, :_D] / sq) * bn2w_ref[...] + bn2b_ref[...]
    out = lax.dot_general(qbn, w2_ref[...], (((1,), (1,)), ((), ())),
                          preferred_element_type=jnp.float32) + b2_ref[...]
    qst_ref[...] = xb + (out - xb)
    lpart = jnp.sum((out - xb) ** 2)

    @pl.when(i == 0)
    def _init():
        counts_ref[...] = jnp.sum(hist_ref[...], axis=0, keepdims=True)
        losssum_ref[0, 0] = lpart

    @pl.when(i != 0)
    def _acc():
        losssum_ref[0, 0] += lpart


def kernel(x, emb, W1, b1, W2, b2, bn1_w, bn1_b, bn2_w, bn2_b):
    shape = x.shape
    T = shape[0] * shape[1]
    xf = x.reshape(T, _IN)
    grid = T // _TB
    full = lambda i: (0, 0)
    idx3 = pl.pallas_call(
        _dist_body,
        grid=(grid,),
        in_specs=[
            pl.BlockSpec((_TB, _IN), lambda i: (i, 0)),
            pl.BlockSpec((_K, _D), full),
            pl.BlockSpec((_D, _IN), full),
            pl.BlockSpec((1, _D), full),
            pl.BlockSpec((1, _IN), full),
            pl.BlockSpec((1, _IN), full),
        ],
        out_specs=pl.BlockSpec((1, 1, _TB), lambda i: (i, 0, 0)),
        out_shape=jax.ShapeDtypeStruct((grid, 1, _TB), jnp.int32),
        scratch_shapes=[pltpu.VMEM((1, _K), jnp.float32)],
    )(xf, emb, W1, b1[None, :], bn1_w[None, :], bn1_b[None, :])

    emb128 = jnp.pad(emb, ((0, 0), (0, 128 - _D)))
    q, hist = _sc_gather_hist(idx3.reshape(T), emb128)

    grid2 = T // _TB2
    qst, counts, losssum = pl.pallas_call(
        _out_body,
        grid=(grid2,),
        in_specs=[
            pl.BlockSpec((_TB2, _IN), lambda i: (i, 0)),
            pl.BlockSpec((_TB2, 128), lambda i: (i, 0)),
            pl.BlockSpec((_IN, _D), full),
            pl.BlockSpec((1, _IN), full),
            pl.BlockSpec((1, _D), full),
            pl.BlockSpec((1, _D), full),
            pl.BlockSpec((2, _K), full),
        ],
        out_specs=[
            pl.BlockSpec((_TB2, _IN), lambda i: (i, 0)),
            pl.BlockSpec((1, _K), full),
            pl.BlockSpec(memory_space=pltpu.SMEM),
        ],
        out_shape=[
            jax.ShapeDtypeStruct((T, _IN), jnp.float32),
            jax.ShapeDtypeStruct((1, _K), jnp.float32),
            jax.ShapeDtypeStruct((1, 1), jnp.float32),
        ],
    )(xf, q, W2, b2[None, :], bn2_w[None, :], bn2_b[None, :], hist)
    m = losssum[0, 0] / jnp.float32(T * _IN)
    loss = m + _CC * m
    usage = counts[0] / jnp.float32(T)
    return (loss, qst.reshape(shape), usage, emb)


# speedup vs baseline: 1.0081x; 1.0081x over previous
import functools

import jax
import jax.numpy as jnp
from jax import lax
from jax.experimental import pallas as pl
from jax.experimental.pallas import tpu as pltpu
from jax.experimental.pallas import tpu_sc as plsc

_K = 8192
_D = 32
_IN = 256
_CC = 0.25
_EPS = 1e-5
_TB = 512
_KC = 4
_TB2 = 512
_NW = 32
_T = 4096
_BPW = _T // _NW


def _dist_body(x_ref, emb_ref, w1_ref, b1_ref, bn1w_ref, bn1b_ref,
               idx_ref, bsq_ref):
    i = pl.program_id(0)

    @pl.when(i == 0)
    def _precompute():
        e = emb_ref[...]
        bsq_ref[...] = jnp.sum(e * e, axis=1)[None, :]

    xb = x_ref[...]
    sq = jnp.sqrt(jnp.float32(1.0 + _EPS))
    flat = (xb / sq) * bn1w_ref[...] + bn1b_ref[...]
    h = lax.dot_general(flat, w1_ref[...], (((1,), (1,)), ((), ())),
                        preferred_element_type=jnp.float32) + b1_ref[...]
    a = jnp.sum(h * h, axis=1, keepdims=True)
    kc = _K // _KC
    minv = None
    idx = None
    for j in range(_KC):
        cj = lax.dot_general(h, emb_ref[pl.ds(j * kc, kc), :],
                             (((1,), (1,)), ((), ())),
                             preferred_element_type=jnp.float32)
        dj = (a + bsq_ref[:, pl.ds(j * kc, kc)]) - 2.0 * cj
        mj = jnp.min(dj, axis=1, keepdims=True)
        io = lax.broadcasted_iota(jnp.int32, (_TB, kc), 1) + j * kc
        ij = jnp.min(jnp.where(dj == mj, io, _K), axis=1)
        if j == 0:
            minv, idx = mj, ij
        else:
            take = (mj < minv)[:, 0]
            idx = jnp.where(take, ij, idx)
            minv = jnp.minimum(minv, mj)
    idx_ref[...] = idx.reshape(1, 1, _TB)


_sc_mesh = plsc.VectorSubcoreMesh(core_axis_name="c", subcore_axis_name="s")


@functools.partial(
    pl.kernel,
    out_type=[
        jax.ShapeDtypeStruct((_T, 128), jnp.float32),
        jax.ShapeDtypeStruct((2, _K), jnp.float32),
    ],
    mesh=_sc_mesh,
    scratch_types=[
        pltpu.VMEM((_BPW,), jnp.int32),
        pltpu.VMEM((_BPW, 128), jnp.float32),
        pltpu.VMEM((_BPW,), jnp.float32),
        pltpu.VMEM((_K,), jnp.float32),
        pltpu.VMEM_SHARED((_K,), jnp.float32),
        pltpu.SemaphoreType.DMA,
    ],
)
def _sc_gather_hist(idx_hbm, emb_hbm, q_hbm, hist_hbm,
                    idx_v, rows_v, ones_v, zbuf_v, shared, sem):
    s = lax.axis_index("s")
    c = lax.axis_index("c")
    wid = s * 2 + c
    base = wid * _BPW
    pltpu.sync_copy(idx_hbm.at[pl.ds(base, _BPW)], idx_v)
    gather = pltpu.async_copy(emb_hbm.at[idx_v], rows_v, sem)

    zeros16 = jnp.zeros((16,), jnp.float32)
    ones16 = jnp.ones((16,), jnp.float32)

    @pl.when(s == 0)
    def _zero_shared():
        def _zero(j, carry):
            zbuf_v[pl.ds(j * 16, 16)] = zeros16
            return carry

        lax.fori_loop(0, _K // 16, _zero, 0)
        pltpu.sync_copy(zbuf_v, shared)

    for j in range(_BPW // 16):
        ones_v[pl.ds(j * 16, 16)] = ones16
    plsc.subcore_barrier()
    pltpu.sync_copy(ones_v, shared.at[idx_v], add=True)
    plsc.subcore_barrier()

    @pl.when(s == 0)
    def _dump_hist():
        pltpu.sync_copy(shared, hist_hbm.at[c])

    gather.wait()
    pltpu.sync_copy(rows_v, q_hbm.at[pl.ds(base, _BPW)])


def _out_body(x_ref, q_ref, w2_ref, b2_ref, bn2w_ref, bn2b_ref, hist_ref,
              qst_ref, counts_ref, losssum_ref):
    i = pl.program_id(0)
    xb = x_ref[...]
    sq = jnp.sqrt(jnp.float32(1.0 + _EPS))
    qbn = (q_ref[:, :_D] / sq) * bn2w_ref[...] + bn2b_ref[...]
    out = lax.dot_general(qbn, w2_ref[...], (((1,), (1,)), ((), ())),
                          preferred_element_type=jnp.float32) + b2_ref[...]
    qst_ref[...] = xb + (out - xb)
    lpart = jnp.sum((out - xb) ** 2)

    @pl.when(i == 0)
    def _init():
        counts_ref[...] = jnp.sum(hist_ref[...], axis=0, keepdims=True)
        losssum_ref[0, 0] = lpart

    @pl.when(i != 0)
    def _acc():
        losssum_ref[0, 0] += lpart


def kernel(x, emb, W1, b1, W2, b2, bn1_w, bn1_b, bn2_w, bn2_b):
    shape = x.shape
    T = shape[0] * shape[1]
    xf = x.reshape(T, _IN)
    grid = T // _TB
    full = lambda i: (0, 0)
    idx3 = pl.pallas_call(
        _dist_body,
        grid=(grid,),
        in_specs=[
            pl.BlockSpec((_TB, _IN), lambda i: (i, 0)),
            pl.BlockSpec((_K, _D), full),
            pl.BlockSpec((_D, _IN), full),
            pl.BlockSpec((1, _D), full),
            pl.BlockSpec((1, _IN), full),
            pl.BlockSpec((1, _IN), full),
        ],
        out_specs=pl.BlockSpec((1, 1, _TB), lambda i: (i, 0, 0)),
        out_shape=jax.ShapeDtypeStruct((grid, 1, _TB), jnp.int32),
        scratch_shapes=[pltpu.VMEM((1, _K), jnp.float32)],
    )(xf, emb, W1, b1[None, :], bn1_w[None, :], bn1_b[None, :])

    emb128 = jnp.pad(emb, ((0, 0), (0, 128 - _D)))
    q, hist = _sc_gather_hist(idx3.reshape(T), emb128)

    grid2 = T // _TB2
    qst, counts, losssum = pl.pallas_call(
        _out_body,
        grid=(grid2,),
        in_specs=[
            pl.BlockSpec((_TB2, _IN), lambda i: (i, 0)),
            pl.BlockSpec((_TB2, 128), lambda i: (i, 0)),
            pl.BlockSpec((_IN, _D), full),
            pl.BlockSpec((1, _IN), full),
            pl.BlockSpec((1, _D), full),
            pl.BlockSpec((1, _D), full),
            pl.BlockSpec((2, _K), full),
        ],
        out_specs=[
            pl.BlockSpec((_TB2, _IN), lambda i: (i, 0)),
            pl.BlockSpec((1, _K), full),
            pl.BlockSpec(memory_space=pltpu.SMEM),
        ],
        out_shape=[
            jax.ShapeDtypeStruct((T, _IN), jnp.float32),
            jax.ShapeDtypeStruct((1, _K), jnp.float32),
            jax.ShapeDtypeStruct((1, 1), jnp.float32),
        ],
    )(xf, q, W2, b2[None, :], bn2_w[None, :], bn2_b[None, :], hist)
    m = losssum[0, 0] / jnp.float32(T * _IN)
    loss = m + _CC * m
    usage = counts[0] / jnp.float32(T)
    return (loss, qst.reshape(shape), usage, emb)
